# baseline (speedup 1.0000x reference)
import functools

import jax
import jax.numpy as jnp
from jax import lax
from jax.experimental import pallas as pl
from jax.experimental.pallas import tpu as pltpu
from jax.experimental.pallas import tpu_sc as plsc

_NC = 2
_NS = 16
_NW = _NC * _NS
_CHUNK = 80
_SUP = 8
_NBUF = 3
_LAG = 2


@functools.lru_cache(maxsize=None)
def _make_scatter(n_pad, n_feat, n_edges):
    edges_per_tile = n_edges // _NW
    n_chunks = edges_per_tile // _CHUNK
    n_sup = -(-n_chunks // _SUP)
    rows_per_tile = n_pad // _NS
    assert edges_per_tile * _NW == n_edges
    assert n_chunks * _CHUNK == edges_per_tile
    assert rows_per_tile * _NS == n_pad and rows_per_tile % _CHUNK == 0

    mesh = plsc.VectorSubcoreMesh(core_axis_name="c", subcore_axis_name="s")

    @functools.partial(
        pl.kernel,
        mesh=mesh,
        out_type=jax.ShapeDtypeStruct((_NC, n_pad, n_feat), jnp.float32),
        scratch_types=[
            pltpu.VMEM_SHARED((n_pad, n_feat), jnp.float32),
            pltpu.VMEM((_SUP, _CHUNK), jnp.int32),
            pltpu.VMEM((_SUP, _CHUNK), jnp.int32),
            pltpu.VMEM((_SUP, _CHUNK), jnp.int32),
            pltpu.VMEM((_SUP, _CHUNK), jnp.int32),
            pltpu.VMEM((_SUP, _CHUNK), jnp.int32),
            pltpu.VMEM((_SUP, _CHUNK), jnp.int32),
            *([pltpu.VMEM((_CHUNK, n_feat), jnp.float32)] * _NBUF),
            *([pltpu.SemaphoreType.DMA] * (2 * _NBUF + 3)),
        ],
    )
    def scatter(x_hbm, row_hbm, col_hbm, out_hbm, acc, *rest):
        rbufs = rest[0:3]
        cbufs = rest[3:6]
        gbufs = rest[6:6 + _NBUF]
        gsems = rest[6 + _NBUF:6 + 2 * _NBUF]
        ssems = rest[6 + 2 * _NBUF:6 + 3 * _NBUF]
        isems = rest[6 + 3 * _NBUF:6 + 3 * _NBUF + 3]
        gbuf0 = gbufs[0]
        c = lax.axis_index("c")
        s = lax.axis_index("s")
        wid = c * _NS + s

        def fetch_idx(sup):
            b = sup % 3
            lo = sup * _SUP
            sz = min(_SUP, n_chunks - lo)
            rdst = rbufs[b] if sz == _SUP else rbufs[b].at[pl.ds(0, sz)]
            cdst = cbufs[b] if sz == _SUP else cbufs[b].at[pl.ds(0, sz)]
            return (
                pltpu.async_copy(row_hbm.at[wid, pl.ds(lo, sz)], rdst,
                                 isems[b]),
                pltpu.async_copy(col_hbm.at[wid, pl.ds(lo, sz)], cdst,
                                 isems[b]),
            )

        pend = fetch_idx(0)

        zero = jnp.zeros((16,), jnp.float32)

        def zrow(i, _):
            def zcol(j, _):
                gbuf0[i, pl.ds(j * 16, 16)] = zero
                return 0
            return lax.fori_loop(0, n_feat // 16, zcol, 0)

        lax.fori_loop(0, _CHUNK, zrow, 0)
        row_base = s * rows_per_tile
        zd = [pltpu.async_copy(
                  gbuf0, acc.at[pl.ds(row_base + k * _CHUNK, _CHUNK)],
                  gsems[k % _NBUF])
              for k in range(rows_per_tile // _CHUNK)]
        for d in zd:
            d.wait()

        gd = [None] * n_chunks
        sd = [None] * n_chunks

        def issue_scatter(t):
            ps, pi = divmod(t, _SUP)
            gd[t].wait()
            sd[t] = pltpu.async_copy(gbufs[t % _NBUF],
                                     acc.at[rbufs[ps % 3].at[pi]],
                                     ssems[t % _NBUF], add=True)

        for t in range(n_chunks):
            sup, i = divmod(t, _SUP)
            if i == 0:
                for p in pend:
                    p.wait()
                if sup + 1 < n_sup:
                    pend = fetch_idx(sup + 1)
            if t >= _NBUF:
                sd[t - _NBUF].wait()
            gd[t] = pltpu.async_copy(x_hbm.at[cbufs[sup % 3].at[i]],
                                     gbufs[t % _NBUF], gsems[t % _NBUF])
            if t == _LAG:
                plsc.subcore_barrier()
            if t >= _LAG:
                issue_scatter(t - _LAG)
        for t in range(max(0, n_chunks - _LAG), n_chunks):
            issue_scatter(t)
        for t in range(max(0, n_chunks - _NBUF), n_chunks):
            sd[t].wait()
        plsc.subcore_barrier()

        pltpu.sync_copy(acc.at[pl.ds(row_base, rows_per_tile)],
                        out_hbm.at[c, pl.ds(row_base, rows_per_tile)])

    return scatter


@functools.lru_cache(maxsize=None)
def _make_combine(n_nodes, n_feat, blk):
    def body(p_ref, w_ref, b_ref, o_ref):
        agg = p_ref[0] + p_ref[1]
        o_ref[...] = jnp.maximum(
            jnp.dot(agg, w_ref[...], preferred_element_type=jnp.float32)
            + b_ref[...], 0.0)

    return pl.pallas_call(
        body,
        grid=(n_nodes // blk,),
        in_specs=[
            pl.BlockSpec((2, blk, n_feat), lambda i: (0, i, 0)),
            pl.BlockSpec((n_feat, n_feat), lambda i: (0, 0)),
            pl.BlockSpec((1, n_feat), lambda i: (0, 0)),
        ],
        out_specs=pl.BlockSpec((blk, n_feat), lambda i: (i, 0)),
        out_shape=jax.ShapeDtypeStruct((n_nodes, n_feat), jnp.float32),
    )


def kernel(x, edge_index, weight, bias):
    n_nodes, in_feat = x.shape
    n_edges = edge_index.shape[1]
    ei = edge_index.astype(jnp.int32)
    rows_per_tile = -(-(n_nodes + 1) // (_NS * _CHUNK)) * _CHUNK
    n_pad = rows_per_tile * _NS
    edges_per_tile = -(-n_edges // (_NW * _CHUNK)) * _CHUNK
    n_edges_pad = edges_per_tile * _NW
    rows, cols = ei[0], ei[1]
    if n_edges_pad != n_edges:
        pad = n_edges_pad - n_edges
        spread = jnp.arange(pad, dtype=jnp.int32)
        rows = jnp.concatenate(
            [rows, n_nodes + spread % (n_pad - n_nodes)])
        cols = jnp.concatenate([cols, spread % n_nodes])
    n_chunks = edges_per_tile // _CHUNK
    row3 = rows.reshape(_NW, n_chunks, _CHUNK)
    col3 = cols.reshape(_NW, n_chunks, _CHUNK)
    partials = _make_scatter(n_pad, in_feat, n_edges_pad)(x, row3, col3)
    return _make_combine(n_nodes, weight.shape[1], 1000)(
        partials, weight, bias.reshape(1, -1))


# speedup vs baseline: 1.0211x; 1.0211x over previous
import functools

import jax
import jax.numpy as jnp
from jax import lax
from jax.experimental import pallas as pl
from jax.experimental.pallas import tpu as pltpu
from jax.experimental.pallas import tpu_sc as plsc

_NC = 2
_NS = 16
_NW = _NC * _NS
_CHUNK = 80
_SUP = 8
_NBUF = 3
_LAG = 2


@functools.lru_cache(maxsize=None)
def _make_scatter(n_pad, n_feat, n_edges):
    edges_per_tile = n_edges // _NW
    n_chunks = edges_per_tile // _CHUNK
    n_sup = -(-n_chunks // _SUP)
    rows_per_tile = n_pad // _NS
    assert edges_per_tile * _NW == n_edges
    assert n_chunks * _CHUNK == edges_per_tile
    assert rows_per_tile * _NS == n_pad and rows_per_tile % _CHUNK == 0

    mesh = plsc.VectorSubcoreMesh(core_axis_name="c", subcore_axis_name="s")

    @functools.partial(
        pl.kernel,
        mesh=mesh,
        out_type=jax.ShapeDtypeStruct((_NC, n_pad, n_feat), jnp.float32),
        scratch_types=[
            pltpu.VMEM_SHARED((n_pad, n_feat), jnp.float32),
            pltpu.VMEM((_SUP, _CHUNK), jnp.int32),
            pltpu.VMEM((_SUP, _CHUNK), jnp.int32),
            pltpu.VMEM((_SUP, _CHUNK), jnp.int32),
            pltpu.VMEM((_SUP, _CHUNK), jnp.int32),
            pltpu.VMEM((_SUP, _CHUNK), jnp.int32),
            pltpu.VMEM((_SUP, _CHUNK), jnp.int32),
            *([pltpu.VMEM((_CHUNK, n_feat), jnp.float32)] * _NBUF),
            *([pltpu.SemaphoreType.DMA] * (2 * _NBUF + 3)),
        ],
    )
    def scatter(x_hbm, row_hbm, col_hbm, out_hbm, acc, *rest):
        rbufs = rest[0:3]
        cbufs = rest[3:6]
        gbufs = rest[6:6 + _NBUF]
        gsems = rest[6 + _NBUF:6 + 2 * _NBUF]
        ssems = rest[6 + 2 * _NBUF:6 + 3 * _NBUF]
        isems = rest[6 + 3 * _NBUF:6 + 3 * _NBUF + 3]
        gbuf0 = gbufs[0]
        c = lax.axis_index("c")
        s = lax.axis_index("s")
        wid = c * _NS + s

        def fetch_idx(sup):
            b = sup % 3
            lo = sup * _SUP
            sz = min(_SUP, n_chunks - lo)
            rdst = rbufs[b] if sz == _SUP else rbufs[b].at[pl.ds(0, sz)]
            cdst = cbufs[b] if sz == _SUP else cbufs[b].at[pl.ds(0, sz)]
            return (
                pltpu.async_copy(row_hbm.at[wid, pl.ds(lo, sz)], rdst,
                                 isems[b]),
                pltpu.async_copy(col_hbm.at[wid, pl.ds(lo, sz)], cdst,
                                 isems[b]),
            )

        pend = fetch_idx(0)

        zero = jnp.zeros((16,), jnp.float32)

        def zrow(i, _):
            def zcol(j, _):
                gbuf0[i, pl.ds(j * 16, 16)] = zero
                return 0
            return lax.fori_loop(0, n_feat // 16, zcol, 0)

        lax.fori_loop(0, _CHUNK, zrow, 0)
        row_base = s * rows_per_tile
        zd = [pltpu.async_copy(
                  gbuf0, acc.at[pl.ds(row_base + k * _CHUNK, _CHUNK)],
                  gsems[k % _NBUF])
              for k in range(rows_per_tile // _CHUNK)]
        for d in zd:
            d.wait()

        gd = [None] * n_chunks
        sd = [None] * n_chunks

        def issue_scatter(t):
            ps, pi = divmod(t, _SUP)
            gd[t].wait()
            sd[t] = pltpu.async_copy(gbufs[t % _NBUF],
                                     acc.at[rbufs[ps % 3].at[pi]],
                                     ssems[t % _NBUF], add=True)

        for t in range(n_chunks):
            sup, i = divmod(t, _SUP)
            if i == 0:
                for p in pend:
                    p.wait()
                if sup + 1 < n_sup:
                    pend = fetch_idx(sup + 1)
            if t >= _NBUF:
                sd[t - _NBUF].wait()
            gd[t] = pltpu.async_copy(x_hbm.at[cbufs[sup % 3].at[i]],
                                     gbufs[t % _NBUF], gsems[t % _NBUF])
            if t == _LAG:
                plsc.subcore_barrier()
            if t >= _LAG:
                issue_scatter(t - _LAG)
        for t in range(max(0, n_chunks - _LAG), n_chunks):
            issue_scatter(t)
        for t in range(max(0, n_chunks - _NBUF), n_chunks):
            sd[t].wait()
        plsc.subcore_barrier()

        pltpu.sync_copy(acc.at[pl.ds(row_base, rows_per_tile)],
                        out_hbm.at[c, pl.ds(row_base, rows_per_tile)])

    return scatter


@functools.lru_cache(maxsize=None)
def _make_combine(n_nodes, n_feat, blk):
    def body(p_ref, w_ref, b_ref, o_ref):
        agg = p_ref[0] + p_ref[1]
        o_ref[...] = jnp.maximum(
            jnp.dot(agg, w_ref[...], preferred_element_type=jnp.float32)
            + b_ref[...], 0.0)

    return pl.pallas_call(
        body,
        grid=(n_nodes // blk,),
        in_specs=[
            pl.BlockSpec((2, blk, n_feat), lambda i: (0, i, 0)),
            pl.BlockSpec((n_feat, n_feat), lambda i: (0, 0)),
            pl.BlockSpec((1, n_feat), lambda i: (0, 0)),
        ],
        out_specs=pl.BlockSpec((blk, n_feat), lambda i: (i, 0)),
        out_shape=jax.ShapeDtypeStruct((n_nodes, n_feat), jnp.float32),
    )


def kernel(x, edge_index, weight, bias):
    n_nodes, in_feat = x.shape
    n_edges = edge_index.shape[1]
    ei = edge_index.astype(jnp.int32)
    rows_per_tile = -(-(n_nodes + 1) // (_NS * _CHUNK)) * _CHUNK
    n_pad = rows_per_tile * _NS
    edges_per_tile = -(-n_edges // (_NW * _CHUNK)) * _CHUNK
    n_edges_pad = edges_per_tile * _NW
    rows, cols = ei[0], ei[1]
    if n_edges_pad != n_edges:
        pad = n_edges_pad - n_edges
        spread = jnp.arange(pad, dtype=jnp.int32)
        rows = jnp.concatenate(
            [rows, n_nodes + spread % (n_pad - n_nodes)])
        cols = jnp.concatenate([cols, spread % n_nodes])
    n_chunks = edges_per_tile // _CHUNK
    row3 = rows.reshape(_NW, n_chunks, _CHUNK)
    col3 = cols.reshape(_NW, n_chunks, _CHUNK)
    partials = _make_scatter(n_pad, in_feat, n_edges_pad)(x, row3, col3)
    return _make_combine(n_nodes, weight.shape[1], 2000)(
        partials, weight, bias.reshape(1, -1))
